# baseline (speedup 1.0000x reference)
import functools

import jax
import jax.numpy as jnp
from jax import lax
from jax.experimental import pallas as pl
from jax.experimental.pallas import tpu as pltpu
from jax.experimental.pallas import tpu_sc as plsc

N_NODES = 10000
N_EDGES = 320000
C_IN = 128
C_HID = 16
C_OUT = 128
W = 32

NC = 2
NS = 16
CHUNK = 128
E_ROWS = N_EDGES // CHUNK
BASE_CH = E_ROWS // (NC * NS)
EXTRA_CH = E_ROWS - BASE_CH * NC * NS
CH_TOT = 80
NBUF = 4
N_ACC = N_NODES + 16
ROWS_PER_SUB = N_NODES // NS
ROW_CH = 125
N_ROW_CH = ROWS_PER_SUB // ROW_CH

BLK = 1000
N_BLK = N_NODES // BLK


def _tc1_body(x_ref, wl_ref, wr_ref, yp_ref, r_ref):
    xb = x_ref[...]
    y = lax.dot_general(
        xb, wl_ref[...], (((1,), (1,)), ((), ())),
        preferred_element_type=jnp.float32,
    )
    yp_ref[...] = jnp.concatenate(
        [y, jnp.ones((BLK, 1), jnp.float32), jnp.zeros((BLK, W - C_HID - 1), jnp.float32)],
        axis=1,
    )
    r_ref[...] = lax.dot_general(
        xb, wr_ref[...], (((1,), (1,)), ((), ())),
        preferred_element_type=jnp.float32,
    )


def _tc1(x, wl, wr):
    return pl.pallas_call(
        _tc1_body,
        grid=(N_BLK,),
        in_specs=[
            pl.BlockSpec((BLK, C_IN), lambda i: (i, 0)),
            pl.BlockSpec((C_HID, C_IN), lambda i: (0, 0)),
            pl.BlockSpec((C_HID, C_IN), lambda i: (0, 0)),
        ],
        out_specs=[
            pl.BlockSpec((BLK, W), lambda i: (i, 0)),
            pl.BlockSpec((BLK, C_HID), lambda i: (i, 0)),
        ],
        out_shape=[
            jax.ShapeDtypeStruct((N_NODES, W), jnp.float32),
            jax.ShapeDtypeStruct((N_NODES, C_HID), jnp.float32),
        ],
    )(x, wl, wr)


def _sc_body(ybig, ei3, out, src_v, dst_v, rows, buf_v, acc_sh, gsems, ssems):
    c = lax.axis_index("c")
    s = lax.axis_index("s")
    g = c * NS + s

    def zrow(i, carry):
        buf_v[i, pl.ds(0, 16)] = jnp.zeros((16,), jnp.float32)
        buf_v[i, pl.ds(16, 16)] = jnp.zeros((16,), jnp.float32)
        return carry

    lax.fori_loop(0, ROW_CH, zrow, 0)

    def zcp(k, carry):
        pltpu.sync_copy(buf_v, acc_sh.at[pl.ds(s * ROWS_PER_SUB + k * ROW_CH, ROW_CH)])
        return carry

    lax.fori_loop(0, N_ROW_CH, zcp, 0)

    pltpu.sync_copy(ei3.at[0, pl.ds(g * BASE_CH, BASE_CH)], src_v.at[pl.ds(0, BASE_CH)])
    pltpu.sync_copy(ei3.at[1, pl.ds(g * BASE_CH, BASE_CH)], dst_v.at[pl.ds(0, BASE_CH)])

    @pl.when(g < EXTRA_CH)
    def _():
        pltpu.sync_copy(
            ei3.at[0, pl.ds(NC * NS * BASE_CH + g, 1)], src_v.at[pl.ds(BASE_CH, 1)]
        )
        pltpu.sync_copy(
            ei3.at[1, pl.ds(NC * NS * BASE_CH + g, 1)], dst_v.at[pl.ds(BASE_CH, 1)]
        )

    n_ch = BASE_CH + jnp.where(g < EXTRA_CH, 1, 0)

    def fill(j, carry):
        for kk in range(CHUNK // 16):
            src_v[j, pl.ds(16 * kk, 16)] = jnp.zeros((16,), jnp.int32)
            dst_v[j, pl.ds(16 * kk, 16)] = jnp.full((16,), N_NODES, jnp.int32)
        return carry

    lax.fori_loop(n_ch, CH_TOT, fill, 0)

    plsc.subcore_barrier()

    for b in range(NBUF):
        pltpu.async_copy(ybig.at[src_v.at[b]], rows.at[b], gsems.at[b])

    def wave(i, carry):
        j = NBUF * i
        for b in range(NBUF):
            pltpu.make_async_copy(ybig.at[src_v.at[0]], rows.at[b], gsems.at[b]).wait()
            pltpu.async_copy(rows.at[b], acc_sh.at[dst_v.at[j + b]], ssems.at[b], add=True)
        for b in range(NBUF):
            pltpu.make_async_copy(rows.at[b], acc_sh.at[dst_v.at[0]], ssems.at[b]).wait()
            jn = jnp.minimum(j + b + NBUF, CH_TOT - 1)
            pltpu.async_copy(ybig.at[src_v.at[jn]], rows.at[b], gsems.at[b])
        return carry

    lax.fori_loop(0, CH_TOT // NBUF, wave, 0)

    for b in range(NBUF):
        pltpu.make_async_copy(ybig.at[src_v.at[0]], rows.at[b], gsems.at[b]).wait()

    plsc.subcore_barrier()

    def ocp(k, carry):
        base = s * ROWS_PER_SUB + k * ROW_CH
        pltpu.sync_copy(acc_sh.at[pl.ds(base, ROW_CH)], buf_v)
        pltpu.sync_copy(buf_v, out.at[c, pl.ds(base, ROW_CH)])
        return carry

    lax.fori_loop(0, N_ROW_CH, ocp, 0)


@functools.cache
def _sc_agg():
    return functools.partial(
        pl.kernel,
        out_type=jax.ShapeDtypeStruct((NC, N_NODES, W), jnp.float32),
        mesh=plsc.VectorSubcoreMesh(
            core_axis_name="c", subcore_axis_name="s", num_cores=NC, num_subcores=NS
        ),
        compiler_params=pltpu.CompilerParams(use_tc_tiling_on_sc=False),
        scratch_types=[
            pltpu.VMEM((CH_TOT, CHUNK), jnp.int32),
            pltpu.VMEM((CH_TOT, CHUNK), jnp.int32),
            pltpu.VMEM((NBUF, CHUNK, W), jnp.float32),
            pltpu.VMEM((ROW_CH, W), jnp.float32),
            pltpu.VMEM_SHARED((N_ACC, W), jnp.float32),
            pltpu.SemaphoreType.DMA((NBUF,)),
            pltpu.SemaphoreType.DMA((NBUF,)),
        ],
    )(_sc_body)


def _tc2_body(accp_ref, r_ref, bl_ref, w2_ref, b2_ref, out_ref):
    ap = accp_ref[...]
    a = ap[0] + ap[1]
    s16 = a[:, :C_HID]
    cnt = a[:, C_HID:C_HID + 1]
    mean = s16 / jnp.maximum(cnt, 1.0)
    h = jnp.maximum(mean + bl_ref[...] + r_ref[...], 0.0)
    out_ref[...] = (
        lax.dot_general(
            h, w2_ref[...], (((1,), (1,)), ((), ())),
            preferred_element_type=jnp.float32,
        )
        + b2_ref[...]
    )


def _tc2(accp, r, bl, w2, b2):
    return pl.pallas_call(
        _tc2_body,
        grid=(N_BLK,),
        in_specs=[
            pl.BlockSpec((NC, BLK, W), lambda i: (0, i, 0)),
            pl.BlockSpec((BLK, C_HID), lambda i: (i, 0)),
            pl.BlockSpec((1, C_HID), lambda i: (0, 0)),
            pl.BlockSpec((C_OUT, C_HID), lambda i: (0, 0)),
            pl.BlockSpec((1, C_OUT), lambda i: (0, 0)),
        ],
        out_specs=pl.BlockSpec((BLK, C_OUT), lambda i: (i, 0)),
        out_shape=jax.ShapeDtypeStruct((N_NODES, C_OUT), jnp.float32),
    )(accp, r, bl, w2, b2)


def kernel(x, edge_index, W_l, b_l, W_r, W2, b2):
    ei3 = edge_index.astype(jnp.int32).reshape(2, N_EDGES // CHUNK, CHUNK)

    ybig, r = _tc1(x, W_l, W_r)
    acc = _sc_agg()(ybig, ei3)
    return _tc2(acc, r, b_l.reshape(1, C_HID), W2, b2.reshape(1, C_OUT))


# speedup vs baseline: 1.2344x; 1.2344x over previous
import functools

import jax
import jax.numpy as jnp
from jax import lax
from jax.experimental import pallas as pl
from jax.experimental.pallas import tpu as pltpu
from jax.experimental.pallas import tpu_sc as plsc

N_NODES = 10000
N_EDGES = 320000
C_IN = 128
C_HID = 16
C_OUT = 128
W = 32

NC = 2
NS = 16
CHUNK = 128
E_ROWS = N_EDGES // CHUNK
BASE_CH = E_ROWS // (NC * NS)
EXTRA_CH = E_ROWS - BASE_CH * NC * NS
CH_TOT = 80
NBUF = 4
N_ACC = N_NODES + 16
ROWS_PER_SUB = N_NODES // NS
ROW_CH = 125
N_ROW_CH = ROWS_PER_SUB // ROW_CH

BLK = 1000
N_BLK = N_NODES // BLK


def _tc1_body(x_ref, wl_ref, wr_ref, yp_ref, r_ref):
    xb = x_ref[...]
    y = lax.dot_general(
        xb, wl_ref[...], (((1,), (1,)), ((), ())),
        preferred_element_type=jnp.float32,
    )
    yp_ref[...] = jnp.concatenate(
        [y, jnp.ones((BLK, 1), jnp.float32), jnp.zeros((BLK, W - C_HID - 1), jnp.float32)],
        axis=1,
    )
    r_ref[...] = lax.dot_general(
        xb, wr_ref[...], (((1,), (1,)), ((), ())),
        preferred_element_type=jnp.float32,
    )


def _tc1(x, wl, wr):
    return pl.pallas_call(
        _tc1_body,
        grid=(N_BLK,),
        in_specs=[
            pl.BlockSpec((BLK, C_IN), lambda i: (i, 0)),
            pl.BlockSpec((C_HID, C_IN), lambda i: (0, 0)),
            pl.BlockSpec((C_HID, C_IN), lambda i: (0, 0)),
        ],
        out_specs=[
            pl.BlockSpec((BLK, W), lambda i: (i, 0)),
            pl.BlockSpec((BLK, C_HID), lambda i: (i, 0)),
        ],
        out_shape=[
            jax.ShapeDtypeStruct((N_NODES, W), jnp.float32),
            jax.ShapeDtypeStruct((N_NODES, C_HID), jnp.float32),
        ],
    )(x, wl, wr)


def _sc_body(ybig, ei3, out, src_v, dst_v, rows_v0, rows_v1, buf_v, acc_sh,
             gsem0, gsem1, ssem0, ssem1):
    c = lax.axis_index("c")
    s = lax.axis_index("s")
    g = c * NS + s

    def zrow(i, carry):
        buf_v[i, pl.ds(0, 16)] = jnp.zeros((16,), jnp.float32)
        buf_v[i, pl.ds(16, 16)] = jnp.zeros((16,), jnp.float32)
        return carry

    lax.fori_loop(0, ROW_CH, zrow, 0)

    def zcp(k, carry):
        pltpu.sync_copy(buf_v, acc_sh.at[pl.ds(s * ROWS_PER_SUB + k * ROW_CH, ROW_CH)])
        return carry

    lax.fori_loop(0, N_ROW_CH, zcp, 0)

    pltpu.sync_copy(ei3.at[0, pl.ds(g * BASE_CH, BASE_CH)], src_v.at[pl.ds(0, BASE_CH)])
    pltpu.sync_copy(ei3.at[1, pl.ds(g * BASE_CH, BASE_CH)], dst_v.at[pl.ds(0, BASE_CH)])

    @pl.when(g < EXTRA_CH)
    def _():
        pltpu.sync_copy(
            ei3.at[0, pl.ds(NC * NS * BASE_CH + g, 1)], src_v.at[pl.ds(BASE_CH, 1)]
        )
        pltpu.sync_copy(
            ei3.at[1, pl.ds(NC * NS * BASE_CH + g, 1)], dst_v.at[pl.ds(BASE_CH, 1)]
        )

    n_ch = BASE_CH + jnp.where(g < EXTRA_CH, 1, 0)

    def fill(j, carry):
        for kk in range(CHUNK // 16):
            src_v[j, pl.ds(16 * kk, 16)] = jnp.zeros((16,), jnp.int32)
            dst_v[j, pl.ds(16 * kk, 16)] = jnp.full((16,), N_NODES, jnp.int32)
        return carry

    lax.fori_loop(n_ch, CH_TOT, fill, 0)

    plsc.subcore_barrier()

    pltpu.async_copy(ybig.at[src_v.at[0]], rows_v0, gsem0)
    pltpu.async_copy(ybig.at[src_v.at[1]], rows_v1, gsem1)

    def pair(i, carry):
        j0 = 2 * i
        j1 = j0 + 1
        pltpu.make_async_copy(ybig.at[src_v.at[0]], rows_v0, gsem0).wait()
        pltpu.async_copy(rows_v0, acc_sh.at[dst_v.at[j0]], ssem0, add=True)
        pltpu.make_async_copy(ybig.at[src_v.at[0]], rows_v1, gsem1).wait()
        pltpu.async_copy(rows_v1, acc_sh.at[dst_v.at[j1]], ssem1, add=True)
        jn0 = jnp.minimum(j0 + 2, CH_TOT - 1)
        jn1 = jnp.minimum(j1 + 2, CH_TOT - 1)
        pltpu.make_async_copy(rows_v0, acc_sh.at[dst_v.at[0]], ssem0).wait()
        pltpu.async_copy(ybig.at[src_v.at[jn0]], rows_v0, gsem0)
        pltpu.make_async_copy(rows_v1, acc_sh.at[dst_v.at[0]], ssem1).wait()
        pltpu.async_copy(ybig.at[src_v.at[jn1]], rows_v1, gsem1)
        return carry

    lax.fori_loop(0, CH_TOT // 2, pair, 0)

    pltpu.make_async_copy(ybig.at[src_v.at[0]], rows_v0, gsem0).wait()
    pltpu.make_async_copy(ybig.at[src_v.at[0]], rows_v1, gsem1).wait()

    plsc.subcore_barrier()

    def ocp(k, carry):
        base = s * ROWS_PER_SUB + k * ROW_CH
        pltpu.sync_copy(acc_sh.at[pl.ds(base, ROW_CH)], buf_v)
        pltpu.sync_copy(buf_v, out.at[c, pl.ds(base, ROW_CH)])
        return carry

    lax.fori_loop(0, N_ROW_CH, ocp, 0)


@functools.cache
def _sc_agg():
    return functools.partial(
        pl.kernel,
        out_type=jax.ShapeDtypeStruct((NC, N_NODES, W), jnp.float32),
        mesh=plsc.VectorSubcoreMesh(
            core_axis_name="c", subcore_axis_name="s", num_cores=NC, num_subcores=NS
        ),
        compiler_params=pltpu.CompilerParams(use_tc_tiling_on_sc=False),
        scratch_types=[
            pltpu.VMEM((CH_TOT, CHUNK), jnp.int32),
            pltpu.VMEM((CH_TOT, CHUNK), jnp.int32),
            pltpu.VMEM((CHUNK, W), jnp.float32),
            pltpu.VMEM((CHUNK, W), jnp.float32),
            pltpu.VMEM((ROW_CH, W), jnp.float32),
            pltpu.VMEM_SHARED((N_ACC, W), jnp.float32),
            pltpu.SemaphoreType.DMA,
            pltpu.SemaphoreType.DMA,
            pltpu.SemaphoreType.DMA,
            pltpu.SemaphoreType.DMA,
        ],
    )(_sc_body)


def _tc2_body(accp_ref, r_ref, bl_ref, w2_ref, b2_ref, out_ref):
    ap = accp_ref[...]
    a = ap[0] + ap[1]
    s16 = a[:, :C_HID]
    cnt = a[:, C_HID:C_HID + 1]
    mean = s16 / jnp.maximum(cnt, 1.0)
    h = jnp.maximum(mean + bl_ref[...] + r_ref[...], 0.0)
    out_ref[...] = (
        lax.dot_general(
            h, w2_ref[...], (((1,), (1,)), ((), ())),
            preferred_element_type=jnp.float32,
        )
        + b2_ref[...]
    )


def _tc2(accp, r, bl, w2, b2):
    return pl.pallas_call(
        _tc2_body,
        grid=(N_BLK,),
        in_specs=[
            pl.BlockSpec((NC, BLK, W), lambda i: (0, i, 0)),
            pl.BlockSpec((BLK, C_HID), lambda i: (i, 0)),
            pl.BlockSpec((1, C_HID), lambda i: (0, 0)),
            pl.BlockSpec((C_OUT, C_HID), lambda i: (0, 0)),
            pl.BlockSpec((1, C_OUT), lambda i: (0, 0)),
        ],
        out_specs=pl.BlockSpec((BLK, C_OUT), lambda i: (i, 0)),
        out_shape=jax.ShapeDtypeStruct((N_NODES, C_OUT), jnp.float32),
    )(accp, r, bl, w2, b2)


def kernel(x, edge_index, W_l, b_l, W_r, W2, b2):
    ei3 = edge_index.astype(jnp.int32).reshape(2, N_EDGES // CHUNK, CHUNK)

    ybig, r = _tc1(x, W_l, W_r)
    acc = _sc_agg()(ybig, ei3)
    return _tc2(acc, r, b_l.reshape(1, C_HID), W2, b2.reshape(1, C_OUT))


# speedup vs baseline: 1.3620x; 1.1034x over previous
import functools

import jax
import jax.numpy as jnp
from jax import lax
from jax.experimental import pallas as pl
from jax.experimental.pallas import tpu as pltpu
from jax.experimental.pallas import tpu_sc as plsc

N_NODES = 10000
N_EDGES = 320000
C_IN = 128
C_HID = 16
C_OUT = 128
W = 32

NC = 2
NS = 16
CHUNK = 128
E_ROWS = N_EDGES // CHUNK
BASE_CH = E_ROWS // (NC * NS)
EXTRA_CH = E_ROWS - BASE_CH * NC * NS
CH_TOT = 80
NBUF = 4
N_ACC = N_NODES + 16
ROWS_PER_SUB = N_NODES // NS
ROW_CH = 125
N_ROW_CH = ROWS_PER_SUB // ROW_CH

BLK = 1000
N_BLK = N_NODES // BLK


def _tc1_body(x_ref, wl_ref, wr_ref, yp_ref, r_ref):
    xb = x_ref[...]
    y = lax.dot_general(
        xb, wl_ref[...], (((1,), (1,)), ((), ())),
        preferred_element_type=jnp.float32,
    )
    yp_ref[...] = jnp.concatenate(
        [y, jnp.ones((BLK, 1), jnp.float32), jnp.zeros((BLK, W - C_HID - 1), jnp.float32)],
        axis=1,
    )
    r_ref[...] = lax.dot_general(
        xb, wr_ref[...], (((1,), (1,)), ((), ())),
        preferred_element_type=jnp.float32,
    )


def _tc1(x, wl, wr):
    return pl.pallas_call(
        _tc1_body,
        grid=(N_BLK,),
        in_specs=[
            pl.BlockSpec((BLK, C_IN), lambda i: (i, 0)),
            pl.BlockSpec((C_HID, C_IN), lambda i: (0, 0)),
            pl.BlockSpec((C_HID, C_IN), lambda i: (0, 0)),
        ],
        out_specs=[
            pl.BlockSpec((BLK, W), lambda i: (i, 0)),
            pl.BlockSpec((BLK, C_HID), lambda i: (i, 0)),
        ],
        out_shape=[
            jax.ShapeDtypeStruct((N_NODES, W), jnp.float32),
            jax.ShapeDtypeStruct((N_NODES, C_HID), jnp.float32),
        ],
    )(x, wl, wr)


def _sc_body(ybig, ei3, out, src_v, dst_v, rows_v0, rows_v1, buf_v, acc_sh,
             gsem0, gsem1, ssem0, ssem1):
    c = lax.axis_index("c")
    s = lax.axis_index("s")
    g = c * NS + s

    def zrow(i, carry):
        buf_v[i, pl.ds(0, 16)] = jnp.zeros((16,), jnp.float32)
        buf_v[i, pl.ds(16, 16)] = jnp.zeros((16,), jnp.float32)
        return carry

    lax.fori_loop(0, ROW_CH, zrow, 0)

    def zcp(k, carry):
        pltpu.sync_copy(buf_v, acc_sh.at[pl.ds(s * ROWS_PER_SUB + k * ROW_CH, ROW_CH)])
        return carry

    lax.fori_loop(0, N_ROW_CH, zcp, 0)

    pltpu.sync_copy(ei3.at[0, pl.ds(g * BASE_CH, BASE_CH)], src_v.at[pl.ds(0, BASE_CH)])
    pltpu.sync_copy(ei3.at[1, pl.ds(g * BASE_CH, BASE_CH)], dst_v.at[pl.ds(0, BASE_CH)])

    @pl.when(g < EXTRA_CH)
    def _():
        pltpu.sync_copy(
            ei3.at[0, pl.ds(NC * NS * BASE_CH + g, 1)], src_v.at[pl.ds(BASE_CH, 1)]
        )
        pltpu.sync_copy(
            ei3.at[1, pl.ds(NC * NS * BASE_CH + g, 1)], dst_v.at[pl.ds(BASE_CH, 1)]
        )

    n_ch = BASE_CH + jnp.where(g < EXTRA_CH, 1, 0)

    def fill(j, carry):
        for kk in range(CHUNK // 16):
            src_v[j, pl.ds(16 * kk, 16)] = jnp.zeros((16,), jnp.int32)
            dst_v[j, pl.ds(16 * kk, 16)] = jnp.full((16,), N_NODES, jnp.int32)
        return carry

    lax.fori_loop(n_ch, CH_TOT, fill, 0)

    plsc.subcore_barrier()

    pltpu.async_copy(ybig.at[src_v.at[0]], rows_v0, gsem0)

    def pair(i, carry):
        j0 = 2 * i
        j1 = j0 + 1
        pltpu.make_async_copy(ybig.at[src_v.at[0]], rows_v0, gsem0).wait()
        pltpu.async_copy(ybig.at[src_v.at[j1]], rows_v1, gsem1)
        pltpu.sync_copy(rows_v0, acc_sh.at[dst_v.at[j0]], add=True)
        pltpu.make_async_copy(ybig.at[src_v.at[0]], rows_v1, gsem1).wait()
        jn = jnp.minimum(j0 + 2, CH_TOT - 1)
        pltpu.async_copy(ybig.at[src_v.at[jn]], rows_v0, gsem0)
        pltpu.sync_copy(rows_v1, acc_sh.at[dst_v.at[j1]], add=True)
        return carry

    lax.fori_loop(0, CH_TOT // 2, pair, 0)

    pltpu.make_async_copy(ybig.at[src_v.at[0]], rows_v0, gsem0).wait()

    plsc.subcore_barrier()

    def ocp(k, carry):
        base = s * ROWS_PER_SUB + k * ROW_CH
        pltpu.sync_copy(acc_sh.at[pl.ds(base, ROW_CH)], buf_v)
        pltpu.sync_copy(buf_v, out.at[c, pl.ds(base, ROW_CH)])
        return carry

    lax.fori_loop(0, N_ROW_CH, ocp, 0)


@functools.cache
def _sc_agg():
    return functools.partial(
        pl.kernel,
        out_type=jax.ShapeDtypeStruct((NC, N_NODES, W), jnp.float32),
        mesh=plsc.VectorSubcoreMesh(
            core_axis_name="c", subcore_axis_name="s", num_cores=NC, num_subcores=NS
        ),
        compiler_params=pltpu.CompilerParams(use_tc_tiling_on_sc=False),
        scratch_types=[
            pltpu.VMEM((CH_TOT, CHUNK), jnp.int32),
            pltpu.VMEM((CH_TOT, CHUNK), jnp.int32),
            pltpu.VMEM((CHUNK, W), jnp.float32),
            pltpu.VMEM((CHUNK, W), jnp.float32),
            pltpu.VMEM((ROW_CH, W), jnp.float32),
            pltpu.VMEM_SHARED((N_ACC, W), jnp.float32),
            pltpu.SemaphoreType.DMA,
            pltpu.SemaphoreType.DMA,
            pltpu.SemaphoreType.DMA,
            pltpu.SemaphoreType.DMA,
        ],
    )(_sc_body)


def _tc2_body(accp_ref, r_ref, bl_ref, w2_ref, b2_ref, out_ref):
    ap = accp_ref[...]
    a = ap[0] + ap[1]
    s16 = a[:, :C_HID]
    cnt = a[:, C_HID:C_HID + 1]
    mean = s16 / jnp.maximum(cnt, 1.0)
    h = jnp.maximum(mean + bl_ref[...] + r_ref[...], 0.0)
    out_ref[...] = (
        lax.dot_general(
            h, w2_ref[...], (((1,), (1,)), ((), ())),
            preferred_element_type=jnp.float32,
        )
        + b2_ref[...]
    )


def _tc2(accp, r, bl, w2, b2):
    return pl.pallas_call(
        _tc2_body,
        grid=(N_BLK,),
        in_specs=[
            pl.BlockSpec((NC, BLK, W), lambda i: (0, i, 0)),
            pl.BlockSpec((BLK, C_HID), lambda i: (i, 0)),
            pl.BlockSpec((1, C_HID), lambda i: (0, 0)),
            pl.BlockSpec((C_OUT, C_HID), lambda i: (0, 0)),
            pl.BlockSpec((1, C_OUT), lambda i: (0, 0)),
        ],
        out_specs=pl.BlockSpec((BLK, C_OUT), lambda i: (i, 0)),
        out_shape=jax.ShapeDtypeStruct((N_NODES, C_OUT), jnp.float32),
    )(accp, r, bl, w2, b2)


def kernel(x, edge_index, W_l, b_l, W_r, W2, b2):
    ei3 = edge_index.astype(jnp.int32).reshape(2, N_EDGES // CHUNK, CHUNK)

    ybig, r = _tc1(x, W_l, W_r)
    acc = _sc_agg()(ybig, ei3)
    return _tc2(acc, r, b_l.reshape(1, C_HID), W2, b2.reshape(1, C_OUT))


# speedup vs baseline: 1.3622x; 1.0001x over previous
import functools

import jax
import jax.numpy as jnp
from jax import lax
from jax.experimental import pallas as pl
from jax.experimental.pallas import tpu as pltpu
from jax.experimental.pallas import tpu_sc as plsc

N_NODES = 10000
N_EDGES = 320000
C_IN = 128
C_HID = 16
C_OUT = 128
W = 32

NC = 2
NS = 16
CHUNK = 128
E_ROWS = N_EDGES // CHUNK
BASE_CH = E_ROWS // (NC * NS)
EXTRA_CH = E_ROWS - BASE_CH * NC * NS
CH_TOT = 80
NBUF = 4
N_ACC = N_NODES + CHUNK
ROWS_PER_SUB = N_NODES // NS
ROW_CH = 125
N_ROW_CH = ROWS_PER_SUB // ROW_CH

BLK = 1000
N_BLK = N_NODES // BLK


def _tc1_body(x_ref, wl_ref, wr_ref, yp_ref, r_ref):
    xb = x_ref[...]
    y = lax.dot_general(
        xb, wl_ref[...], (((1,), (1,)), ((), ())),
        preferred_element_type=jnp.float32,
    )
    yp_ref[...] = jnp.concatenate(
        [y, jnp.ones((BLK, 1), jnp.float32), jnp.zeros((BLK, W - C_HID - 1), jnp.float32)],
        axis=1,
    )
    r_ref[...] = lax.dot_general(
        xb, wr_ref[...], (((1,), (1,)), ((), ())),
        preferred_element_type=jnp.float32,
    )


def _tc1(x, wl, wr):
    return pl.pallas_call(
        _tc1_body,
        grid=(N_BLK,),
        in_specs=[
            pl.BlockSpec((BLK, C_IN), lambda i: (i, 0)),
            pl.BlockSpec((C_HID, C_IN), lambda i: (0, 0)),
            pl.BlockSpec((C_HID, C_IN), lambda i: (0, 0)),
        ],
        out_specs=[
            pl.BlockSpec((BLK, W), lambda i: (i, 0)),
            pl.BlockSpec((BLK, C_HID), lambda i: (i, 0)),
        ],
        out_shape=[
            jax.ShapeDtypeStruct((N_NODES, W), jnp.float32),
            jax.ShapeDtypeStruct((N_NODES, C_HID), jnp.float32),
        ],
    )(x, wl, wr)


def _sc_body(ybig, ei3, out, src_v, dst_v, rows_v0, rows_v1, buf_v, acc_sh,
             gsem0, gsem1, ssem0, ssem1):
    c = lax.axis_index("c")
    s = lax.axis_index("s")
    g = c * NS + s

    def zrow(i, carry):
        buf_v[i, pl.ds(0, 16)] = jnp.zeros((16,), jnp.float32)
        buf_v[i, pl.ds(16, 16)] = jnp.zeros((16,), jnp.float32)
        return carry

    lax.fori_loop(0, ROW_CH, zrow, 0)

    def zcp(k, carry):
        pltpu.sync_copy(buf_v, acc_sh.at[pl.ds(s * ROWS_PER_SUB + k * ROW_CH, ROW_CH)])
        return carry

    lax.fori_loop(0, N_ROW_CH, zcp, 0)

    pltpu.sync_copy(ei3.at[0, pl.ds(g * BASE_CH, BASE_CH)], src_v.at[pl.ds(0, BASE_CH)])
    pltpu.sync_copy(ei3.at[1, pl.ds(g * BASE_CH, BASE_CH)], dst_v.at[pl.ds(0, BASE_CH)])

    @pl.when(g < EXTRA_CH)
    def _():
        pltpu.sync_copy(
            ei3.at[0, pl.ds(NC * NS * BASE_CH + g, 1)], src_v.at[pl.ds(BASE_CH, 1)]
        )
        pltpu.sync_copy(
            ei3.at[1, pl.ds(NC * NS * BASE_CH + g, 1)], dst_v.at[pl.ds(BASE_CH, 1)]
        )

    n_ch = BASE_CH + jnp.where(g < EXTRA_CH, 1, 0)

    def fill(j, carry):
        for kk in range(CHUNK // 16):
            src_v[j, pl.ds(16 * kk, 16)] = jnp.zeros((16,), jnp.int32)
            dst_v[j, pl.ds(16 * kk, 16)] = (
                jnp.full((16,), N_NODES + 16 * kk, jnp.int32)
                + lax.iota(jnp.int32, 16)
            )
        return carry

    lax.fori_loop(n_ch, CH_TOT, fill, 0)

    plsc.subcore_barrier()

    pltpu.async_copy(ybig.at[src_v.at[0]], rows_v0, gsem0)

    def pair(i, carry):
        j0 = 2 * i
        j1 = j0 + 1
        pltpu.make_async_copy(ybig.at[src_v.at[0]], rows_v0, gsem0).wait()
        pltpu.async_copy(ybig.at[src_v.at[j1]], rows_v1, gsem1)
        pltpu.sync_copy(rows_v0, acc_sh.at[dst_v.at[j0]], add=True)
        pltpu.make_async_copy(ybig.at[src_v.at[0]], rows_v1, gsem1).wait()
        jn = jnp.minimum(j0 + 2, CH_TOT - 1)
        pltpu.async_copy(ybig.at[src_v.at[jn]], rows_v0, gsem0)
        pltpu.sync_copy(rows_v1, acc_sh.at[dst_v.at[j1]], add=True)
        return carry

    lax.fori_loop(0, CH_TOT // 2, pair, 0)

    pltpu.make_async_copy(ybig.at[src_v.at[0]], rows_v0, gsem0).wait()

    plsc.subcore_barrier()

    def ocp(k, carry):
        base = s * ROWS_PER_SUB + k * ROW_CH
        pltpu.sync_copy(acc_sh.at[pl.ds(base, ROW_CH)], buf_v)
        pltpu.sync_copy(buf_v, out.at[c, pl.ds(base, ROW_CH)])
        return carry

    lax.fori_loop(0, N_ROW_CH, ocp, 0)


@functools.cache
def _sc_agg():
    return functools.partial(
        pl.kernel,
        out_type=jax.ShapeDtypeStruct((NC, N_NODES, W), jnp.float32),
        mesh=plsc.VectorSubcoreMesh(
            core_axis_name="c", subcore_axis_name="s", num_cores=NC, num_subcores=NS
        ),
        compiler_params=pltpu.CompilerParams(use_tc_tiling_on_sc=False),
        scratch_types=[
            pltpu.VMEM((CH_TOT, CHUNK), jnp.int32),
            pltpu.VMEM((CH_TOT, CHUNK), jnp.int32),
            pltpu.VMEM((CHUNK, W), jnp.float32),
            pltpu.VMEM((CHUNK, W), jnp.float32),
            pltpu.VMEM((ROW_CH, W), jnp.float32),
            pltpu.VMEM_SHARED((N_ACC, W), jnp.float32),
            pltpu.SemaphoreType.DMA,
            pltpu.SemaphoreType.DMA,
            pltpu.SemaphoreType.DMA,
            pltpu.SemaphoreType.DMA,
        ],
    )(_sc_body)


def _tc2_body(accp_ref, r_ref, bl_ref, w2_ref, b2_ref, out_ref):
    ap = accp_ref[...]
    a = ap[0] + ap[1]
    s16 = a[:, :C_HID]
    cnt = a[:, C_HID:C_HID + 1]
    mean = s16 / jnp.maximum(cnt, 1.0)
    h = jnp.maximum(mean + bl_ref[...] + r_ref[...], 0.0)
    out_ref[...] = (
        lax.dot_general(
            h, w2_ref[...], (((1,), (1,)), ((), ())),
            preferred_element_type=jnp.float32,
        )
        + b2_ref[...]
    )


def _tc2(accp, r, bl, w2, b2):
    return pl.pallas_call(
        _tc2_body,
        grid=(N_BLK,),
        in_specs=[
            pl.BlockSpec((NC, BLK, W), lambda i: (0, i, 0)),
            pl.BlockSpec((BLK, C_HID), lambda i: (i, 0)),
            pl.BlockSpec((1, C_HID), lambda i: (0, 0)),
            pl.BlockSpec((C_OUT, C_HID), lambda i: (0, 0)),
            pl.BlockSpec((1, C_OUT), lambda i: (0, 0)),
        ],
        out_specs=pl.BlockSpec((BLK, C_OUT), lambda i: (i, 0)),
        out_shape=jax.ShapeDtypeStruct((N_NODES, C_OUT), jnp.float32),
    )(accp, r, bl, w2, b2)


def kernel(x, edge_index, W_l, b_l, W_r, W2, b2):
    ei3 = edge_index.astype(jnp.int32).reshape(2, N_EDGES // CHUNK, CHUNK)

    ybig, r = _tc1(x, W_l, W_r)
    acc = _sc_agg()(ybig, ei3)
    return _tc2(acc, r, b_l.reshape(1, C_HID), W2, b2.reshape(1, C_OUT))


# speedup vs baseline: 2.5219x; 1.8514x over previous
import functools

import jax
import jax.numpy as jnp
from jax import lax
from jax.experimental import pallas as pl
from jax.experimental.pallas import tpu as pltpu
from jax.experimental.pallas import tpu_sc as plsc

N_NODES = 10000
N_EDGES = 320000
C_IN = 128
C_HID = 16
C_OUT = 128
W = 32

NC = 2
NS = 16
CHUNK = 128
E_ROWS = N_EDGES // CHUNK
BASE_CH = E_ROWS // (NC * NS)
EXTRA_CH = E_ROWS - BASE_CH * NC * NS
CH_TOT = 80
NBUF = 4
N_ACC = N_NODES + CHUNK
ROWS_PER_SUB = N_NODES // NS
ROW_CH = 125
N_ROW_CH = ROWS_PER_SUB // ROW_CH

BLK = 1000
N_BLK = N_NODES // BLK


def _tc1_body(x_ref, wl_ref, wr_ref, yp_ref, r_ref):
    xb = x_ref[...]
    y = lax.dot_general(
        xb, wl_ref[...], (((1,), (1,)), ((), ())),
        preferred_element_type=jnp.float32,
    )
    yp_ref[...] = jnp.concatenate(
        [y, jnp.ones((BLK, 1), jnp.float32), jnp.zeros((BLK, W - C_HID - 1), jnp.float32)],
        axis=1,
    )
    r_ref[...] = lax.dot_general(
        xb, wr_ref[...], (((1,), (1,)), ((), ())),
        preferred_element_type=jnp.float32,
    )


def _tc1(x, wl, wr):
    return pl.pallas_call(
        _tc1_body,
        grid=(N_BLK,),
        in_specs=[
            pl.BlockSpec((BLK, C_IN), lambda i: (i, 0)),
            pl.BlockSpec((C_HID, C_IN), lambda i: (0, 0)),
            pl.BlockSpec((C_HID, C_IN), lambda i: (0, 0)),
        ],
        out_specs=[
            pl.BlockSpec((BLK, W), lambda i: (i, 0)),
            pl.BlockSpec((BLK, C_HID), lambda i: (i, 0)),
        ],
        out_shape=[
            jax.ShapeDtypeStruct((N_NODES, W), jnp.float32),
            jax.ShapeDtypeStruct((N_NODES, C_HID), jnp.float32),
        ],
    )(x, wl, wr)


def _sc_body(ybig, ei3, out, src_v, dst_v, rows_v0, rows_v1, buf_v, acc_sh,
             gsem0, gsem1):
    c = lax.axis_index("c")
    s = lax.axis_index("s")
    g = c * NS + s

    def zrow(i, carry):
        buf_v[i, pl.ds(0, 16)] = jnp.zeros((16,), jnp.float32)
        buf_v[i, pl.ds(16, 16)] = jnp.zeros((16,), jnp.float32)
        return carry

    lax.fori_loop(0, ROW_CH, zrow, 0)

    def zcp(k, carry):
        pltpu.sync_copy(buf_v, acc_sh.at[pl.ds(s * ROWS_PER_SUB + k * ROW_CH, ROW_CH)])
        return carry

    lax.fori_loop(0, N_ROW_CH, zcp, 0)

    pltpu.sync_copy(ei3.at[0, pl.ds(g * BASE_CH, BASE_CH)], src_v.at[pl.ds(0, BASE_CH)])
    pltpu.sync_copy(ei3.at[1, pl.ds(g * BASE_CH, BASE_CH)], dst_v.at[pl.ds(0, BASE_CH)])

    @pl.when(g < EXTRA_CH)
    def _():
        pltpu.sync_copy(
            ei3.at[0, pl.ds(NC * NS * BASE_CH + g, 1)], src_v.at[pl.ds(BASE_CH, 1)]
        )
        pltpu.sync_copy(
            ei3.at[1, pl.ds(NC * NS * BASE_CH + g, 1)], dst_v.at[pl.ds(BASE_CH, 1)]
        )

    n_ch = BASE_CH + jnp.where(g < EXTRA_CH, 1, 0)

    plsc.subcore_barrier()

    pltpu.async_copy(ybig.at[src_v.at[0]], rows_v0, gsem0)

    def pair(i, carry):
        j0 = 2 * i
        j1 = j0 + 1
        pltpu.make_async_copy(ybig.at[src_v.at[j0]], rows_v0, gsem0).wait()
        pltpu.async_copy(ybig.at[src_v.at[j1]], rows_v1, gsem1)
        pltpu.sync_copy(rows_v0, acc_sh.at[dst_v.at[j0]], add=True)
        pltpu.make_async_copy(ybig.at[src_v.at[j1]], rows_v1, gsem1).wait()
        jn = jnp.minimum(j0 + 2, n_ch - 1)
        pltpu.async_copy(ybig.at[src_v.at[jn]], rows_v0, gsem0)
        pltpu.sync_copy(rows_v1, acc_sh.at[dst_v.at[j1]], add=True)
        return carry

    lax.fori_loop(0, BASE_CH // 2, pair, 0)

    pltpu.make_async_copy(ybig.at[src_v.at[BASE_CH]], rows_v0, gsem0).wait()

    @pl.when(g < EXTRA_CH)
    def _():
        pltpu.sync_copy(rows_v0, acc_sh.at[dst_v.at[BASE_CH]], add=True)

    plsc.subcore_barrier()

    def ocp(k, carry):
        base = s * ROWS_PER_SUB + k * ROW_CH
        pltpu.sync_copy(acc_sh.at[pl.ds(base, ROW_CH)], buf_v)
        pltpu.sync_copy(buf_v, out.at[c, pl.ds(base, ROW_CH)])
        return carry

    lax.fori_loop(0, N_ROW_CH, ocp, 0)


@functools.cache
def _sc_agg():
    return functools.partial(
        pl.kernel,
        out_type=jax.ShapeDtypeStruct((NC, N_NODES, W), jnp.float32),
        mesh=plsc.VectorSubcoreMesh(
            core_axis_name="c", subcore_axis_name="s", num_cores=NC, num_subcores=NS
        ),
        compiler_params=pltpu.CompilerParams(use_tc_tiling_on_sc=False),
        scratch_types=[
            pltpu.VMEM((BASE_CH + 1, CHUNK), jnp.int32),
            pltpu.VMEM((BASE_CH + 1, CHUNK), jnp.int32),
            pltpu.VMEM((CHUNK, W), jnp.float32),
            pltpu.VMEM((CHUNK, W), jnp.float32),
            pltpu.VMEM((ROW_CH, W), jnp.float32),
            pltpu.VMEM_SHARED((N_NODES, W), jnp.float32),
            pltpu.SemaphoreType.DMA,
            pltpu.SemaphoreType.DMA,
        ],
    )(_sc_body)


def _tc2_body(accp_ref, r_ref, bl_ref, w2_ref, b2_ref, out_ref):
    ap = accp_ref[...]
    a = ap[0] + ap[1]
    s16 = a[:, :C_HID]
    cnt = a[:, C_HID:C_HID + 1]
    mean = s16 / jnp.maximum(cnt, 1.0)
    h = jnp.maximum(mean + bl_ref[...] + r_ref[...], 0.0)
    out_ref[...] = (
        lax.dot_general(
            h, w2_ref[...], (((1,), (1,)), ((), ())),
            preferred_element_type=jnp.float32,
        )
        + b2_ref[...]
    )


def _tc2(accp, r, bl, w2, b2):
    return pl.pallas_call(
        _tc2_body,
        grid=(N_BLK,),
        in_specs=[
            pl.BlockSpec((NC, BLK, W), lambda i: (0, i, 0)),
            pl.BlockSpec((BLK, C_HID), lambda i: (i, 0)),
            pl.BlockSpec((1, C_HID), lambda i: (0, 0)),
            pl.BlockSpec((C_OUT, C_HID), lambda i: (0, 0)),
            pl.BlockSpec((1, C_OUT), lambda i: (0, 0)),
        ],
        out_specs=pl.BlockSpec((BLK, C_OUT), lambda i: (i, 0)),
        out_shape=jax.ShapeDtypeStruct((N_NODES, C_OUT), jnp.float32),
    )(accp, r, bl, w2, b2)


def kernel(x, edge_index, W_l, b_l, W_r, W2, b2):
    ei3 = edge_index.astype(jnp.int32).reshape(2, N_EDGES // CHUNK, CHUNK)

    ybig, r = _tc1(x, W_l, W_r)
    acc = _sc_agg()(ybig, ei3)
    return _tc2(acc, r, b_l.reshape(1, C_HID), W2, b2.reshape(1, C_OUT))


# speedup vs baseline: 3.1529x; 1.2502x over previous
import functools

import jax
import jax.numpy as jnp
from jax import lax
from jax.experimental import pallas as pl
from jax.experimental.pallas import tpu as pltpu
from jax.experimental.pallas import tpu_sc as plsc

N_NODES = 10000
N_EDGES = 320000
C_IN = 128
C_HID = 16
C_OUT = 128
W = 32

NC = 2
NS = 16
CHUNK = 128
E_ROWS = N_EDGES // CHUNK
BASE_CH = E_ROWS // (NC * NS)
EXTRA_CH = E_ROWS - BASE_CH * NC * NS
CH_TOT = 80
NBUF = 4
N_ACC = N_NODES + CHUNK
ROWS_PER_SUB = N_NODES // NS
ROW_CH = 125
N_ROW_CH = ROWS_PER_SUB // ROW_CH

BLK = 1000
N_BLK = N_NODES // BLK


def _tc1_body(x_ref, wl_ref, wr_ref, yp_ref, r_ref):
    xb = x_ref[...]
    y = lax.dot_general(
        xb, wl_ref[...], (((1,), (1,)), ((), ())),
        preferred_element_type=jnp.float32,
    )
    yp_ref[...] = jnp.concatenate(
        [y, jnp.ones((BLK, 1), jnp.float32), jnp.zeros((BLK, W - C_HID - 1), jnp.float32)],
        axis=1,
    )
    r_ref[...] = lax.dot_general(
        xb, wr_ref[...], (((1,), (1,)), ((), ())),
        preferred_element_type=jnp.float32,
    )


def _tc1(x, wl, wr):
    return pl.pallas_call(
        _tc1_body,
        grid=(N_BLK,),
        in_specs=[
            pl.BlockSpec((BLK, C_IN), lambda i: (i, 0)),
            pl.BlockSpec((C_HID, C_IN), lambda i: (0, 0)),
            pl.BlockSpec((C_HID, C_IN), lambda i: (0, 0)),
        ],
        out_specs=[
            pl.BlockSpec((BLK, W), lambda i: (i, 0)),
            pl.BlockSpec((BLK, C_HID), lambda i: (i, 0)),
        ],
        out_shape=[
            jax.ShapeDtypeStruct((N_NODES, W), jnp.float32),
            jax.ShapeDtypeStruct((N_NODES, C_HID), jnp.float32),
        ],
    )(x, wl, wr)


def _sc_body(ybig, ei3, out, src_v, dst_v, rows_v0, rows_v1, buf_v, acc_sh,
             ybig_sh, gsem0, gsem1):
    c = lax.axis_index("c")
    s = lax.axis_index("s")
    g = c * NS + s

    def zrow(i, carry):
        buf_v[i, pl.ds(0, 16)] = jnp.zeros((16,), jnp.float32)
        buf_v[i, pl.ds(16, 16)] = jnp.zeros((16,), jnp.float32)
        return carry

    lax.fori_loop(0, ROW_CH, zrow, 0)

    def zcp(k, carry):
        pltpu.sync_copy(buf_v, acc_sh.at[pl.ds(s * ROWS_PER_SUB + k * ROW_CH, ROW_CH)])
        return carry

    lax.fori_loop(0, N_ROW_CH, zcp, 0)

    def ycp(k, carry):
        base = s * ROWS_PER_SUB + k * ROW_CH
        pltpu.sync_copy(ybig.at[pl.ds(base, ROW_CH)], ybig_sh.at[pl.ds(base, ROW_CH)])
        return carry

    lax.fori_loop(0, N_ROW_CH, ycp, 0)

    pltpu.sync_copy(ei3.at[0, pl.ds(g * BASE_CH, BASE_CH)], src_v.at[pl.ds(0, BASE_CH)])
    pltpu.sync_copy(ei3.at[1, pl.ds(g * BASE_CH, BASE_CH)], dst_v.at[pl.ds(0, BASE_CH)])

    @pl.when(g < EXTRA_CH)
    def _():
        pltpu.sync_copy(
            ei3.at[0, pl.ds(NC * NS * BASE_CH + g, 1)], src_v.at[pl.ds(BASE_CH, 1)]
        )
        pltpu.sync_copy(
            ei3.at[1, pl.ds(NC * NS * BASE_CH + g, 1)], dst_v.at[pl.ds(BASE_CH, 1)]
        )

    n_ch = BASE_CH + jnp.where(g < EXTRA_CH, 1, 0)

    plsc.subcore_barrier()

    pltpu.async_copy(ybig_sh.at[src_v.at[0]], rows_v0, gsem0)

    def pair(i, carry):
        j0 = 2 * i
        j1 = j0 + 1
        pltpu.make_async_copy(ybig_sh.at[src_v.at[j0]], rows_v0, gsem0).wait()
        pltpu.async_copy(ybig_sh.at[src_v.at[j1]], rows_v1, gsem1)
        pltpu.sync_copy(rows_v0, acc_sh.at[dst_v.at[j0]], add=True)
        pltpu.make_async_copy(ybig_sh.at[src_v.at[j1]], rows_v1, gsem1).wait()
        jn = jnp.minimum(j0 + 2, n_ch - 1)
        pltpu.async_copy(ybig_sh.at[src_v.at[jn]], rows_v0, gsem0)
        pltpu.sync_copy(rows_v1, acc_sh.at[dst_v.at[j1]], add=True)
        return carry

    lax.fori_loop(0, BASE_CH // 2, pair, 0)

    pltpu.make_async_copy(ybig_sh.at[src_v.at[BASE_CH]], rows_v0, gsem0).wait()

    @pl.when(g < EXTRA_CH)
    def _():
        pltpu.sync_copy(rows_v0, acc_sh.at[dst_v.at[BASE_CH]], add=True)

    plsc.subcore_barrier()

    def ocp(k, carry):
        base = s * ROWS_PER_SUB + k * ROW_CH
        pltpu.sync_copy(acc_sh.at[pl.ds(base, ROW_CH)], buf_v)
        pltpu.sync_copy(buf_v, out.at[c, pl.ds(base, ROW_CH)])
        return carry

    lax.fori_loop(0, N_ROW_CH, ocp, 0)


@functools.cache
def _sc_agg():
    return functools.partial(
        pl.kernel,
        out_type=jax.ShapeDtypeStruct((NC, N_NODES, W), jnp.float32),
        mesh=plsc.VectorSubcoreMesh(
            core_axis_name="c", subcore_axis_name="s", num_cores=NC, num_subcores=NS
        ),
        compiler_params=pltpu.CompilerParams(use_tc_tiling_on_sc=False),
        scratch_types=[
            pltpu.VMEM((BASE_CH + 1, CHUNK), jnp.int32),
            pltpu.VMEM((BASE_CH + 1, CHUNK), jnp.int32),
            pltpu.VMEM((CHUNK, W), jnp.float32),
            pltpu.VMEM((CHUNK, W), jnp.float32),
            pltpu.VMEM((ROW_CH, W), jnp.float32),
            pltpu.VMEM_SHARED((N_NODES, W), jnp.float32),
            pltpu.VMEM_SHARED((N_NODES, W), jnp.float32),
            pltpu.SemaphoreType.DMA,
            pltpu.SemaphoreType.DMA,
        ],
    )(_sc_body)


def _tc2_body(accp_ref, r_ref, bl_ref, w2_ref, b2_ref, out_ref):
    ap = accp_ref[...]
    a = ap[0] + ap[1]
    s16 = a[:, :C_HID]
    cnt = a[:, C_HID:C_HID + 1]
    mean = s16 / jnp.maximum(cnt, 1.0)
    h = jnp.maximum(mean + bl_ref[...] + r_ref[...], 0.0)
    out_ref[...] = (
        lax.dot_general(
            h, w2_ref[...], (((1,), (1,)), ((), ())),
            preferred_element_type=jnp.float32,
        )
        + b2_ref[...]
    )


def _tc2(accp, r, bl, w2, b2):
    return pl.pallas_call(
        _tc2_body,
        grid=(N_BLK,),
        in_specs=[
            pl.BlockSpec((NC, BLK, W), lambda i: (0, i, 0)),
            pl.BlockSpec((BLK, C_HID), lambda i: (i, 0)),
            pl.BlockSpec((1, C_HID), lambda i: (0, 0)),
            pl.BlockSpec((C_OUT, C_HID), lambda i: (0, 0)),
            pl.BlockSpec((1, C_OUT), lambda i: (0, 0)),
        ],
        out_specs=pl.BlockSpec((BLK, C_OUT), lambda i: (i, 0)),
        out_shape=jax.ShapeDtypeStruct((N_NODES, C_OUT), jnp.float32),
    )(accp, r, bl, w2, b2)


def kernel(x, edge_index, W_l, b_l, W_r, W2, b2):
    ei3 = edge_index.astype(jnp.int32).reshape(2, N_EDGES // CHUNK, CHUNK)

    ybig, r = _tc1(x, W_l, W_r)
    acc = _sc_agg()(ybig, ei3)
    return _tc2(acc, r, b_l.reshape(1, C_HID), W2, b2.reshape(1, C_OUT))
